# Initial kernel scaffold; baseline (speedup 1.0000x reference)
import jax
import jax.numpy as jnp
from jax.experimental import pallas as pl


def kernel(x, table, W, b):
    raise NotImplementedError("write your pallas kernel here")


# speedup vs baseline: 12.9462x; 12.9462x over previous
import functools

import jax
import jax.numpy as jnp
from jax import lax
from jax.experimental import pallas as pl
from jax.experimental.pallas import tpu as pltpu
from jax.experimental.pallas import tpu_sc as plsc

VOCAB = 1000000
DIM = 32
BATCH = 16384
FIELDS = 26

TOTAL = BATCH * FIELDS
CHUNK = 128


def _sc_gather(table, idx2d, *, num_workers, chunks_per_worker):
    mesh = plsc.VectorSubcoreMesh(core_axis_name="c", subcore_axis_name="s")
    rows_per_worker = chunks_per_worker * CHUNK

    @functools.partial(
        pl.kernel,
        mesh=mesh,
        compiler_params=pltpu.CompilerParams(use_tc_tiling_on_sc=False),
        out_type=jax.ShapeDtypeStruct((TOTAL, DIM), jnp.float32),
        scratch_types=[
            pltpu.VMEM((chunks_per_worker, CHUNK), jnp.int32),
            pltpu.VMEM((CHUNK, DIM), jnp.float32),
            pltpu.VMEM((CHUNK, DIM), jnp.float32),
            pltpu.SemaphoreType.DMA,
            pltpu.SemaphoreType.DMA,
        ],
    )
    def k(table_hbm, idx_hbm, out_hbm, idx_v, buf0, buf1, sem0, sem1):
        nc = 2
        wid = lax.axis_index("s") * nc + lax.axis_index("c")
        chunk_base = wid * chunks_per_worker
        row_base = wid * rows_per_worker
        pltpu.sync_copy(idx_hbm.at[pl.ds(chunk_base, chunks_per_worker)], idx_v)

        bufs = (buf0, buf1)
        sems = (sem0, sem1)

        cp0 = pltpu.async_copy(table_hbm.at[idx_v.at[0]], buf0, sem0)

        def body(j, carry):
            del carry
            @pl.when(j + 1 < chunks_per_worker)
            def _():
                for par in range(2):
                    @pl.when((j + 1) % 2 == par)
                    def _():
                        pltpu.async_copy(
                            table_hbm.at[idx_v.at[j + 1]], bufs[par], sems[par]
                        )

            for par in range(2):
                @pl.when(j % 2 == par)
                def _():
                    pltpu.make_async_copy(
                        table_hbm.at[idx_v.at[j]], bufs[par], sems[par]
                    ).wait()
                    pltpu.sync_copy(
                        bufs[par], out_hbm.at[pl.ds(row_base + j * CHUNK, CHUNK)]
                    )
            return 0

        del cp0
        lax.fori_loop(0, chunks_per_worker, body, 0, unroll=2)

    return k(table, idx2d)


def _tc_transform(emb128, w4, b4):
    rows = emb128.shape[0]
    block = 4096
    grid = rows // block

    def body(e_ref, w_ref, b_ref, o_ref):
        acc = jnp.dot(e_ref[...], w_ref[...], preferred_element_type=jnp.float32)
        o_ref[...] = jnp.maximum(acc + b_ref[...], 0.0)

    return pl.pallas_call(
        body,
        grid=(grid,),
        in_specs=[
            pl.BlockSpec((block, 128), lambda i: (i, 0)),
            pl.BlockSpec((128, 128), lambda i: (0, 0)),
            pl.BlockSpec((1, 128), lambda i: (0, 0)),
        ],
        out_specs=pl.BlockSpec((block, 128), lambda i: (i, 0)),
        out_shape=jax.ShapeDtypeStruct((rows, 128), jnp.float32),
    )(emb128, w4, b4)


def kernel(x, table, W, b):
    num_workers = 32
    chunks_per_worker = TOTAL // (num_workers * CHUNK)
    idx2d = x.reshape(num_workers * chunks_per_worker, CHUNK).astype(jnp.int32)
    emb = _sc_gather(table, idx2d, num_workers=num_workers,
                     chunks_per_worker=chunks_per_worker)
    w4 = jnp.kron(jnp.eye(4, dtype=W.dtype), W.T)
    b4 = jnp.tile(b, 4).reshape(1, 128)
    y = _tc_transform(emb.reshape(TOTAL * DIM // 128, 128), w4, b4)
    return y.reshape(BATCH, FIELDS, DIM)
